# baseline (speedup 1.0000x reference)
import functools

import jax
import jax.numpy as jnp
from jax import lax
from jax.experimental import pallas as pl
from jax.experimental.pallas import tpu as pltpu
from jax.experimental.pallas import tpu_sc as plsc

N = 10000
E = 320000
D = 128
DE = 16
H = 128
HH = H // 2

NC = 2
NS = 16
CHUNK = 128
CHUNKS_PER_TILE = 160
E_PAD = NS * CHUNKS_PER_TILE * CHUNK
N_PAD = 10112
ROWS_PER_TILE = N_PAD // NS
NBUF = 2


def _sc_message_pass_body(h_lo, h_hi, e_lo, e_hi, src_hbm, dst_hbm, zeros_hbm,
                          out_hbm, src_all, dst_all, rows_v, e_v, agg_sh,
                          gsem, esem, ssem):
    c = lax.axis_index("c")
    s = lax.axis_index("s")
    base = s * ROWS_PER_TILE
    ebase = s * CHUNKS_PER_TILE

    pltpu.sync_copy(src_hbm.at[s], src_all)
    pltpu.sync_copy(dst_hbm.at[s], dst_all)
    pltpu.sync_copy(zeros_hbm.at[pl.ds(base, ROWS_PER_TILE)],
                    agg_sh.at[pl.ds(base, ROWS_PER_TILE)])
    plsc.subcore_barrier()

    def run_half(h_hbm, e_hbm):
        def fetch(k, b):
            pltpu.async_copy(h_hbm.at[src_all.at[k]], rows_v[b], gsem[b])
            pltpu.async_copy(e_hbm.at[pl.ds((ebase + k) * CHUNK, CHUNK)],
                             e_v[b], esem[b])

        def step(k, b):
            pltpu.make_async_copy(h_hbm.at[src_all.at[k]], rows_v[b],
                                  gsem[b]).wait()
            pltpu.make_async_copy(e_hbm.at[pl.ds(0, CHUNK)], e_v[b],
                                  esem[b]).wait()

            def row_body(r, carry2):
                for j in range(HH // 16):
                    sl = pl.ds(j * 16, 16)
                    rows_v[b][r, sl] = jnp.maximum(
                        rows_v[b][r, sl] + e_v[b][r, sl], 0.0)
                return carry2

            lax.fori_loop(0, CHUNK, row_body, 0, unroll=False)
            pltpu.async_copy(rows_v[b], agg_sh.at[dst_all.at[k]], ssem[b],
                             add=True)
            nb = (b + NBUF - 1) % NBUF

            @pl.when(k >= 1)
            def _():
                pltpu.make_async_copy(rows_v[nb], agg_sh.at[dst_all.at[k]],
                                      ssem[nb]).wait()

            @pl.when(k + NBUF - 1 < CHUNKS_PER_TILE)
            def _():
                fetch(k + NBUF - 1, nb)

        for k in range(NBUF - 1):
            fetch(k, k)

        def outer_body(kk, carry):
            for j in range(NBUF):
                step(kk * NBUF + j, j)
            return carry

        lax.fori_loop(0, CHUNKS_PER_TILE // NBUF, outer_body, 0,
                      unroll=False)
        lb = (CHUNKS_PER_TILE - 1) % NBUF
        pltpu.make_async_copy(rows_v[lb], agg_sh.at[dst_all.at[0]],
                              ssem[lb]).wait()

    @pl.when(c == 0)
    def _():
        run_half(h_lo, e_lo)

    @pl.when(c == 1)
    def _():
        run_half(h_hi, e_hi)

    plsc.subcore_barrier()
    pltpu.sync_copy(agg_sh.at[pl.ds(base, ROWS_PER_TILE)],
                    out_hbm.at[c, pl.ds(base, ROWS_PER_TILE)])


def _sc_message_pass(h_lo, h_hi, e_lo, e_hi, src_t, dst_t, zeros):
    mesh = plsc.VectorSubcoreMesh(core_axis_name="c", subcore_axis_name="s")
    fn = pl.kernel(
        _sc_message_pass_body,
        out_type=jax.ShapeDtypeStruct((NC, N_PAD, HH), jnp.float32),
        mesh=mesh,
        scratch_types=[
            pltpu.VMEM((CHUNKS_PER_TILE, CHUNK), jnp.int32),
            pltpu.VMEM((CHUNKS_PER_TILE, CHUNK), jnp.int32),
            [pltpu.VMEM((CHUNK, HH), jnp.float32)] * NBUF,
            [pltpu.VMEM((CHUNK, HH), jnp.float32)] * NBUF,
            pltpu.VMEM_SHARED((N_PAD, HH), jnp.float32),
            [pltpu.SemaphoreType.DMA] * NBUF,
            [pltpu.SemaphoreType.DMA] * NBUF,
            [pltpu.SemaphoreType.DMA] * NBUF,
        ],
        compiler_params=pltpu.CompilerParams(use_tc_tiling_on_sc=False),
    )
    return fn(h_lo, h_hi, e_lo, e_hi, src_t, dst_t, zeros)


def _edge_mlp_body(ea_ref, w_ref, b_ref, *o_refs):
    v = jnp.dot(ea_ref[...], w_ref[...],
                preferred_element_type=jnp.float32) + b_ref[...]
    for t in range(6):
        o_refs[t][...] = v[:, t * HH:(t + 1) * HH]


def _edge_mlp(edge_attr, w_cat, b_cat):
    BE = 4096
    grid = (E_PAD // BE,)
    out = jax.ShapeDtypeStruct((E_PAD, HH), jnp.float32)
    return pl.pallas_call(
        _edge_mlp_body,
        grid=grid,
        in_specs=[
            pl.BlockSpec((BE, DE), lambda i: (i, 0)),
            pl.BlockSpec((DE, 3 * H), lambda i: (0, 0)),
            pl.BlockSpec((1, 3 * H), lambda i: (0, 0)),
        ],
        out_specs=[pl.BlockSpec((BE, HH), lambda i: (i, 0))] * 6,
        out_shape=[out] * 6,
    )(edge_attr, w_cat, b_cat)


def _node_mlp_body(hlo_ref, hhi_ref, part_ref, w1_ref, b1_ref, w2_ref, b2_ref,
                   lng_ref, lnb_ref, eps_ref, olo_ref, ohi_ref, *, residual):
    h = jnp.concatenate([hlo_ref[...], hhi_ref[...]], axis=-1)
    agg = jnp.concatenate([part_ref[0], part_ref[1]], axis=-1)
    z = (1.0 + eps_ref[0]) * h + agg
    z1 = jnp.dot(z, w1_ref[...], preferred_element_type=jnp.float32)
    z1 = jnp.maximum(z1 + b1_ref[...], 0.0)
    z2 = jnp.dot(z1, w2_ref[...], preferred_element_type=jnp.float32)
    z2 = z2 + b2_ref[...]
    mu = jnp.mean(z2, axis=-1, keepdims=True)
    var = jnp.mean((z2 - mu) ** 2, axis=-1, keepdims=True)
    zn = (z2 - mu) * lax.rsqrt(var + 1e-5) * lng_ref[...] + lnb_ref[...]
    zr = jnp.maximum(zn, 0.0)
    if residual:
        zr = h + 0.3 * zr
    olo_ref[...] = zr[:, :HH]
    ohi_ref[...] = zr[:, HH:]


def _node_mlp(h_lo, h_hi, part, w1, b1, w2, b2, lng, lnb, eps, residual):
    BN = 1000
    grid = (N // BN,)
    body = functools.partial(_node_mlp_body, residual=residual)
    out = jax.ShapeDtypeStruct((N, HH), jnp.float32)
    return pl.pallas_call(
        body,
        grid=grid,
        in_specs=[
            pl.BlockSpec((BN, HH), lambda i: (i, 0)),
            pl.BlockSpec((BN, HH), lambda i: (i, 0)),
            pl.BlockSpec((NC, BN, HH), lambda i: (0, i, 0)),
            pl.BlockSpec((H, 2 * H), lambda i: (0, 0)),
            pl.BlockSpec((1, 2 * H), lambda i: (0, 0)),
            pl.BlockSpec((2 * H, H), lambda i: (0, 0)),
            pl.BlockSpec((1, H), lambda i: (0, 0)),
            pl.BlockSpec((1, H), lambda i: (0, 0)),
            pl.BlockSpec((1, H), lambda i: (0, 0)),
            pl.BlockSpec(memory_space=pltpu.SMEM),
        ],
        out_specs=[pl.BlockSpec((BN, HH), lambda i: (i, 0))] * 2,
        out_shape=[out, out],
    )(h_lo, h_hi, part, w1, b1, w2, b2, lng, lnb, eps)


def kernel(x, edge_index, edge_attr,
           W_edge_0, b_edge_0, eps_0, W1_0, b1_0, bn1_g_0, bn1_b_0,
           W2_0, b2_0, bn_g_0, bn_b_0, ln_g_0, ln_b_0,
           W_edge_1, b_edge_1, eps_1, W1_1, b1_1, bn1_g_1, bn1_b_1,
           W2_1, b2_1, bn_g_1, bn_b_1, ln_g_1, ln_b_1,
           W_edge_2, b_edge_2, eps_2, W1_2, b1_2, bn1_g_2, bn1_b_2,
           W2_2, b2_2, bn_g_2, bn_b_2, ln_g_2, ln_b_2):
    bn_scale = 1.0 / jnp.sqrt(1.0 + 1e-5)
    src_t = jnp.concatenate(
        [edge_index[0], jnp.zeros((E_PAD - E,), jnp.int32)]
    ).reshape(NS, CHUNKS_PER_TILE, CHUNK)
    dst_t = jnp.concatenate(
        [edge_index[1], jnp.full((E_PAD - E,), N, jnp.int32)]
    ).reshape(NS, CHUNKS_PER_TILE, CHUNK)
    ea_pad = jnp.concatenate(
        [edge_attr, jnp.zeros((E_PAD - E, DE), jnp.float32)])
    zeros = jnp.zeros((N_PAD, HH), jnp.float32)

    Ws, Es = [], []
    for (W_e, b_e, eps, W1, b1, g1, bb1, W2, b2, g2, bb2, lg, lb) in (
        (W_edge_0, b_edge_0, eps_0, W1_0, b1_0, bn1_g_0, bn1_b_0, W2_0, b2_0,
         bn_g_0, bn_b_0, ln_g_0, ln_b_0),
        (W_edge_1, b_edge_1, eps_1, W1_1, b1_1, bn1_g_1, bn1_b_1, W2_1, b2_1,
         bn_g_1, bn_b_1, ln_g_1, ln_b_1),
        (W_edge_2, b_edge_2, eps_2, W1_2, b1_2, bn1_g_2, bn1_b_2, W2_2, b2_2,
         bn_g_2, bn_b_2, ln_g_2, ln_b_2),
    ):
        s1 = bn_scale * g1
        w1f = W1 * s1[None, :]
        b1f = (b1 * s1 + bb1)[None, :]
        s2 = bn_scale * g2
        w2f = W2 * s2[None, :]
        b2f = (b2 * s2 + bb2)[None, :]
        Ws.append((eps.reshape(1), w1f, b1f, w2f, b2f,
                   lg[None, :], lb[None, :]))
        Es.append((W_e, b_e))

    w_cat = jnp.concatenate([Es[0][0], Es[1][0], Es[2][0]], axis=1)
    b_cat = jnp.concatenate([Es[0][1], Es[1][1], Es[2][1]])[None, :]
    e_halves = _edge_mlp(ea_pad, w_cat, b_cat)

    h_lo = x[:, :HH]
    h_hi = x[:, HH:]
    for i in range(3):
        eps, w1f, b1f, w2f, b2f, lg, lb = Ws[i]
        e_lo, e_hi = e_halves[2 * i], e_halves[2 * i + 1]
        part = _sc_message_pass(h_lo, h_hi, e_lo, e_hi, src_t, dst_t,
                                zeros)[:, :N]
        h_lo, h_hi = _node_mlp(h_lo, h_hi, part, w1f, b1f, w2f, b2f, lg, lb,
                               eps, residual=(i == 1))
    return jnp.concatenate([h_lo, h_hi], axis=-1)


# speedup vs baseline: 1.0160x; 1.0160x over previous
import functools

import jax
import jax.numpy as jnp
from jax import lax
from jax.experimental import pallas as pl
from jax.experimental.pallas import tpu as pltpu
from jax.experimental.pallas import tpu_sc as plsc

N = 10000
E = 320000
D = 128
DE = 16
H = 128

NC = 2
NS = 16
CHUNK = 64
CHUNKS_PER_TILE = 160
NPASS = 4
PASS_CHUNKS = CHUNKS_PER_TILE // NPASS
E_PAD = NC * NS * CHUNKS_PER_TILE * CHUNK
N_PAD = 10112
ROWS_PER_TILE = N_PAD // NS
NBUF = 2


def _sc_message_pass_body(h_hbm, e_hbm, src_hbm, dst_hbm, zeros_hbm,
                          out_hbm, src_all, dst_all, rows_v, e_v, agg_sh,
                          gsem, esem, ssem):
    c = lax.axis_index("c")
    s = lax.axis_index("s")
    wid = c * NS + s
    base = s * ROWS_PER_TILE

    pltpu.sync_copy(zeros_hbm.at[pl.ds(base, ROWS_PER_TILE)],
                    agg_sh.at[pl.ds(base, ROWS_PER_TILE)])
    plsc.subcore_barrier()

    for p in range(NPASS):
        pltpu.sync_copy(src_hbm.at[wid, pl.ds(p * PASS_CHUNKS, PASS_CHUNKS)],
                        src_all)
        pltpu.sync_copy(dst_hbm.at[wid, pl.ds(p * PASS_CHUNKS, PASS_CHUNKS)],
                        dst_all)

        def fetch(k, b):
            pltpu.async_copy(h_hbm.at[src_all.at[k]], rows_v[b], gsem[b])
            eoff = ((wid * CHUNKS_PER_TILE + p * PASS_CHUNKS) + k) * CHUNK
            pltpu.async_copy(e_hbm.at[pl.ds(eoff, CHUNK)], e_v[b], esem[b])

        def step(k, b):
            pltpu.make_async_copy(h_hbm.at[src_all.at[k]], rows_v[b],
                                  gsem[b]).wait()
            pltpu.make_async_copy(e_hbm.at[pl.ds(0, CHUNK)], e_v[b],
                                  esem[b]).wait()

            def row_body(r, carry2):
                for j in range(H // 16):
                    sl = pl.ds(j * 16, 16)
                    rows_v[b][r, sl] = jnp.maximum(
                        rows_v[b][r, sl] + e_v[b][r, sl], 0.0)
                return carry2

            lax.fori_loop(0, CHUNK, row_body, 0, unroll=False)
            pltpu.async_copy(rows_v[b], agg_sh.at[dst_all.at[k]], ssem[b],
                             add=True)
            nb = (b + 1) % NBUF

            @pl.when(k >= 1)
            def _():
                pltpu.make_async_copy(rows_v[nb], agg_sh.at[dst_all.at[k]],
                                      ssem[nb]).wait()

            @pl.when(k + 1 < PASS_CHUNKS)
            def _():
                fetch(k + 1, nb)

        fetch(0, 0)

        def outer_body(kk, carry):
            for j in range(NBUF):
                step(kk * NBUF + j, j)
            return carry

        lax.fori_loop(0, PASS_CHUNKS // NBUF, outer_body, 0, unroll=False)
        lb = (PASS_CHUNKS - 1) % NBUF
        pltpu.make_async_copy(rows_v[lb], agg_sh.at[dst_all.at[0]],
                              ssem[lb]).wait()

    plsc.subcore_barrier()
    pltpu.sync_copy(agg_sh.at[pl.ds(base, ROWS_PER_TILE)],
                    out_hbm.at[c, pl.ds(base, ROWS_PER_TILE)])


def _sc_message_pass(h, e, src_t, dst_t, zeros):
    mesh = plsc.VectorSubcoreMesh(core_axis_name="c", subcore_axis_name="s")
    fn = pl.kernel(
        _sc_message_pass_body,
        out_type=jax.ShapeDtypeStruct((NC, N_PAD, H), jnp.float32),
        mesh=mesh,
        scratch_types=[
            pltpu.VMEM((PASS_CHUNKS, CHUNK), jnp.int32),
            pltpu.VMEM((PASS_CHUNKS, CHUNK), jnp.int32),
            [pltpu.VMEM((CHUNK, H), jnp.float32)] * NBUF,
            [pltpu.VMEM((CHUNK, H), jnp.float32)] * NBUF,
            pltpu.VMEM_SHARED((N_PAD, H), jnp.float32),
            [pltpu.SemaphoreType.DMA] * NBUF,
            [pltpu.SemaphoreType.DMA] * NBUF,
            [pltpu.SemaphoreType.DMA] * NBUF,
        ],
    )
    return fn(h, e, src_t, dst_t, zeros)


def _edge_mlp_body(ea_ref, w_ref, b_ref, o0_ref, o1_ref, o2_ref):
    v = jnp.dot(ea_ref[...], w_ref[...],
                preferred_element_type=jnp.float32) + b_ref[...]
    o0_ref[...] = v[:, :H]
    o1_ref[...] = v[:, H:2 * H]
    o2_ref[...] = v[:, 2 * H:]


def _edge_mlp(edge_attr, w_cat, b_cat):
    BE = 4096
    grid = (E_PAD // BE,)
    out = jax.ShapeDtypeStruct((E_PAD, H), jnp.float32)
    return pl.pallas_call(
        _edge_mlp_body,
        grid=grid,
        in_specs=[
            pl.BlockSpec((BE, DE), lambda i: (i, 0)),
            pl.BlockSpec((DE, 3 * H), lambda i: (0, 0)),
            pl.BlockSpec((1, 3 * H), lambda i: (0, 0)),
        ],
        out_specs=[pl.BlockSpec((BE, H), lambda i: (i, 0))] * 3,
        out_shape=[out] * 3,
    )(edge_attr, w_cat, b_cat)


def _node_mlp_body(h_ref, part_ref, w1_ref, b1_ref, w2_ref, b2_ref,
                   lng_ref, lnb_ref, eps_ref, o_ref, *, residual):
    h = h_ref[...]
    agg = part_ref[0] + part_ref[1]
    z = (1.0 + eps_ref[0]) * h + agg
    z1 = jnp.dot(z, w1_ref[...], preferred_element_type=jnp.float32)
    z1 = jnp.maximum(z1 + b1_ref[...], 0.0)
    z2 = jnp.dot(z1, w2_ref[...], preferred_element_type=jnp.float32)
    z2 = z2 + b2_ref[...]
    mu = jnp.mean(z2, axis=-1, keepdims=True)
    var = jnp.mean((z2 - mu) ** 2, axis=-1, keepdims=True)
    zn = (z2 - mu) * lax.rsqrt(var + 1e-5) * lng_ref[...] + lnb_ref[...]
    zr = jnp.maximum(zn, 0.0)
    if residual:
        o_ref[...] = h + 0.3 * zr
    else:
        o_ref[...] = zr


def _node_mlp(h, part, w1, b1, w2, b2, lng, lnb, eps, residual):
    BN = 1000
    grid = (N // BN,)
    body = functools.partial(_node_mlp_body, residual=residual)
    return pl.pallas_call(
        body,
        grid=grid,
        in_specs=[
            pl.BlockSpec((BN, H), lambda i: (i, 0)),
            pl.BlockSpec((NC, BN, H), lambda i: (0, i, 0)),
            pl.BlockSpec((H, 2 * H), lambda i: (0, 0)),
            pl.BlockSpec((1, 2 * H), lambda i: (0, 0)),
            pl.BlockSpec((2 * H, H), lambda i: (0, 0)),
            pl.BlockSpec((1, H), lambda i: (0, 0)),
            pl.BlockSpec((1, H), lambda i: (0, 0)),
            pl.BlockSpec((1, H), lambda i: (0, 0)),
            pl.BlockSpec(memory_space=pltpu.SMEM),
        ],
        out_specs=pl.BlockSpec((BN, H), lambda i: (i, 0)),
        out_shape=jax.ShapeDtypeStruct((N, H), jnp.float32),
    )(h, part, w1, b1, w2, b2, lng, lnb, eps)


def kernel(x, edge_index, edge_attr,
           W_edge_0, b_edge_0, eps_0, W1_0, b1_0, bn1_g_0, bn1_b_0,
           W2_0, b2_0, bn_g_0, bn_b_0, ln_g_0, ln_b_0,
           W_edge_1, b_edge_1, eps_1, W1_1, b1_1, bn1_g_1, bn1_b_1,
           W2_1, b2_1, bn_g_1, bn_b_1, ln_g_1, ln_b_1,
           W_edge_2, b_edge_2, eps_2, W1_2, b1_2, bn1_g_2, bn1_b_2,
           W2_2, b2_2, bn_g_2, bn_b_2, ln_g_2, ln_b_2):
    bn_scale = 1.0 / jnp.sqrt(1.0 + 1e-5)
    src_t = jnp.concatenate(
        [edge_index[0], jnp.zeros((E_PAD - E,), jnp.int32)]
    ).reshape(NC * NS, CHUNKS_PER_TILE, CHUNK)
    dst_t = jnp.concatenate(
        [edge_index[1], jnp.full((E_PAD - E,), N, jnp.int32)]
    ).reshape(NC * NS, CHUNKS_PER_TILE, CHUNK)
    ea_pad = jnp.concatenate(
        [edge_attr, jnp.zeros((E_PAD - E, DE), jnp.float32)])
    zeros = jnp.zeros((N_PAD, H), jnp.float32)

    Ws, Es = [], []
    for (W_e, b_e, eps, W1, b1, g1, bb1, W2, b2, g2, bb2, lg, lb) in (
        (W_edge_0, b_edge_0, eps_0, W1_0, b1_0, bn1_g_0, bn1_b_0, W2_0, b2_0,
         bn_g_0, bn_b_0, ln_g_0, ln_b_0),
        (W_edge_1, b_edge_1, eps_1, W1_1, b1_1, bn1_g_1, bn1_b_1, W2_1, b2_1,
         bn_g_1, bn_b_1, ln_g_1, ln_b_1),
        (W_edge_2, b_edge_2, eps_2, W1_2, b1_2, bn1_g_2, bn1_b_2, W2_2, b2_2,
         bn_g_2, bn_b_2, ln_g_2, ln_b_2),
    ):
        s1 = bn_scale * g1
        w1f = W1 * s1[None, :]
        b1f = (b1 * s1 + bb1)[None, :]
        s2 = bn_scale * g2
        w2f = W2 * s2[None, :]
        b2f = (b2 * s2 + bb2)[None, :]
        Ws.append((eps.reshape(1), w1f, b1f, w2f, b2f,
                   lg[None, :], lb[None, :]))
        Es.append((W_e, b_e))

    w_cat = jnp.concatenate([Es[0][0], Es[1][0], Es[2][0]], axis=1)
    b_cat = jnp.concatenate([Es[0][1], Es[1][1], Es[2][1]])[None, :]
    e0, e1, e2 = _edge_mlp(ea_pad, w_cat, b_cat)

    h = x
    for i, e in enumerate((e0, e1, e2)):
        eps, w1f, b1f, w2f, b2f, lg, lb = Ws[i]
        part = _sc_message_pass(h, e, src_t, dst_t, zeros)[:, :N]
        h = _node_mlp(h, part, w1f, b1f, w2f, b2f, lg, lb, eps,
                      residual=(i == 1))
    return h


# speedup vs baseline: 1.0493x; 1.0328x over previous
import functools

import jax
import jax.numpy as jnp
from jax import lax
from jax.experimental import pallas as pl
from jax.experimental.pallas import tpu as pltpu
from jax.experimental.pallas import tpu_sc as plsc

N = 10000
E = 320000
D = 128
DE = 16
H = 128

NC = 2
NS = 16
CHUNK = 128
CHUNKS_PER_TILE = 80
E_PAD = NC * NS * CHUNKS_PER_TILE * CHUNK
N_PAD = 10112
ROWS_PER_TILE = N_PAD // NS
NIB = 4


def _sc_message_pass_body(h_hbm, e_hbm, sd_hbm, zeros_hbm,
                          out_hbm, ib, rows_v, e_v, agg_sh,
                          gsem, esem, ssem, isem):
    c = lax.axis_index("c")
    s = lax.axis_index("s")
    wid = c * NS + s
    base = s * ROWS_PER_TILE

    pltpu.sync_copy(zeros_hbm.at[pl.ds(base, ROWS_PER_TILE)],
                    agg_sh.at[pl.ds(base, ROWS_PER_TILE)])
    plsc.subcore_barrier()

    def fetch_idx_async(k, j):
        pltpu.async_copy(sd_hbm.at[wid, k], ib[j], isem[j])

    def fetch_gather(ij, b):
        pltpu.async_copy(h_hbm.at[ib[ij].at[0]], rows_v[b], gsem[b])

    def fetch_e(k):
        eoff = (wid * CHUNKS_PER_TILE + k) * CHUNK
        pltpu.async_copy(e_hbm.at[pl.ds(eoff, CHUNK)], e_v, esem)

    pltpu.sync_copy(sd_hbm.at[wid, 0], ib[0])
    fetch_idx_async(1, 1)
    fetch_idx_async(2, 2)
    fetch_gather(0, 0)
    fetch_e(0)

    def step(k, j):
        b = j % 2
        nb = (b + 1) % 2
        pltpu.make_async_copy(h_hbm.at[ib[0].at[0]], rows_v[b],
                              gsem[b]).wait()
        pltpu.make_async_copy(e_hbm.at[pl.ds(0, CHUNK)], e_v, esem).wait()

        def row_body(r, carry2):
            for jj in range(H // 16):
                sl = pl.ds(jj * 16, 16)
                rows_v[b][r, sl] = jnp.maximum(
                    rows_v[b][r, sl] + e_v[r, sl], 0.0)
            return carry2

        lax.fori_loop(0, CHUNK, row_body, 0, unroll=False)
        pltpu.async_copy(rows_v[b], agg_sh.at[ib[j].at[1]], ssem[b],
                         add=True)

        @pl.when(k + 1 < CHUNKS_PER_TILE)
        def _():
            fetch_e(k + 1)
            pltpu.make_async_copy(sd_hbm.at[wid, 0], ib[(j + 1) % NIB],
                                  isem[(j + 1) % NIB]).wait()

            @pl.when(k >= 1)
            def _():
                pltpu.make_async_copy(rows_v[nb],
                                      agg_sh.at[ib[0].at[1]],
                                      ssem[nb]).wait()
                fetch_gather((j + 1) % NIB, nb)

                @pl.when(k + 2 < CHUNKS_PER_TILE)
                def _():
                    fetch_idx_async(k + 2, (j + 2) % NIB)

            @pl.when(k == 0)
            def _():
                fetch_gather((j + 1) % NIB, nb)

        return None

    def outer_body(kk, carry):
        for j in range(NIB):
            step(kk * NIB + j, j)
        return carry

    lax.fori_loop(0, CHUNKS_PER_TILE // NIB, outer_body, 0, unroll=False)
    for b in range(2):
        pltpu.make_async_copy(rows_v[b], agg_sh.at[ib[0].at[1]],
                              ssem[b]).wait()

    plsc.subcore_barrier()
    pltpu.sync_copy(agg_sh.at[pl.ds(base, ROWS_PER_TILE)],
                    out_hbm.at[c, pl.ds(base, ROWS_PER_TILE)])


def _sc_message_pass(h, e, sd_t, zeros):
    mesh = plsc.VectorSubcoreMesh(core_axis_name="c", subcore_axis_name="s")
    fn = pl.kernel(
        _sc_message_pass_body,
        out_type=jax.ShapeDtypeStruct((NC, N_PAD, H), jnp.float32),
        mesh=mesh,
        scratch_types=[
            [pltpu.VMEM((2, CHUNK), jnp.int32)] * NIB,
            [pltpu.VMEM((CHUNK, H), jnp.float32)] * 2,
            pltpu.VMEM((CHUNK, H), jnp.float32),
            pltpu.VMEM_SHARED((N_PAD, H), jnp.float32),
            [pltpu.SemaphoreType.DMA] * 2,
            pltpu.SemaphoreType.DMA,
            [pltpu.SemaphoreType.DMA] * 2,
            [pltpu.SemaphoreType.DMA] * NIB,
        ],
    )
    return fn(h, e, sd_t, zeros)


def _edge_mlp_body(ea_ref, w_ref, b_ref, o0_ref, o1_ref, o2_ref):
    v = jnp.dot(ea_ref[...], w_ref[...],
                preferred_element_type=jnp.float32) + b_ref[...]
    o0_ref[...] = v[:, :H]
    o1_ref[...] = v[:, H:2 * H]
    o2_ref[...] = v[:, 2 * H:]


def _edge_mlp(edge_attr, w_cat, b_cat):
    BE = 4096
    grid = (E_PAD // BE,)
    out = jax.ShapeDtypeStruct((E_PAD, H), jnp.float32)
    return pl.pallas_call(
        _edge_mlp_body,
        grid=grid,
        in_specs=[
            pl.BlockSpec((BE, DE), lambda i: (i, 0)),
            pl.BlockSpec((DE, 3 * H), lambda i: (0, 0)),
            pl.BlockSpec((1, 3 * H), lambda i: (0, 0)),
        ],
        out_specs=[pl.BlockSpec((BE, H), lambda i: (i, 0))] * 3,
        out_shape=[out] * 3,
    )(edge_attr, w_cat, b_cat)


def _node_mlp_body(h_ref, part_ref, w1_ref, b1_ref, w2_ref, b2_ref,
                   lng_ref, lnb_ref, eps_ref, o_ref, *, residual):
    h = h_ref[...]
    agg = part_ref[0] + part_ref[1]
    z = (1.0 + eps_ref[0]) * h + agg
    z1 = jnp.dot(z, w1_ref[...], preferred_element_type=jnp.float32)
    z1 = jnp.maximum(z1 + b1_ref[...], 0.0)
    z2 = jnp.dot(z1, w2_ref[...], preferred_element_type=jnp.float32)
    z2 = z2 + b2_ref[...]
    mu = jnp.mean(z2, axis=-1, keepdims=True)
    var = jnp.mean((z2 - mu) ** 2, axis=-1, keepdims=True)
    zn = (z2 - mu) * lax.rsqrt(var + 1e-5) * lng_ref[...] + lnb_ref[...]
    zr = jnp.maximum(zn, 0.0)
    if residual:
        o_ref[...] = h + 0.3 * zr
    else:
        o_ref[...] = zr


def _node_mlp(h, part, w1, b1, w2, b2, lng, lnb, eps, residual):
    BN = 1000
    grid = (N // BN,)
    body = functools.partial(_node_mlp_body, residual=residual)
    return pl.pallas_call(
        body,
        grid=grid,
        in_specs=[
            pl.BlockSpec((BN, H), lambda i: (i, 0)),
            pl.BlockSpec((NC, BN, H), lambda i: (0, i, 0)),
            pl.BlockSpec((H, 2 * H), lambda i: (0, 0)),
            pl.BlockSpec((1, 2 * H), lambda i: (0, 0)),
            pl.BlockSpec((2 * H, H), lambda i: (0, 0)),
            pl.BlockSpec((1, H), lambda i: (0, 0)),
            pl.BlockSpec((1, H), lambda i: (0, 0)),
            pl.BlockSpec((1, H), lambda i: (0, 0)),
            pl.BlockSpec(memory_space=pltpu.SMEM),
        ],
        out_specs=pl.BlockSpec((BN, H), lambda i: (i, 0)),
        out_shape=jax.ShapeDtypeStruct((N, H), jnp.float32),
    )(h, part, w1, b1, w2, b2, lng, lnb, eps)


def kernel(x, edge_index, edge_attr,
           W_edge_0, b_edge_0, eps_0, W1_0, b1_0, bn1_g_0, bn1_b_0,
           W2_0, b2_0, bn_g_0, bn_b_0, ln_g_0, ln_b_0,
           W_edge_1, b_edge_1, eps_1, W1_1, b1_1, bn1_g_1, bn1_b_1,
           W2_1, b2_1, bn_g_1, bn_b_1, ln_g_1, ln_b_1,
           W_edge_2, b_edge_2, eps_2, W1_2, b1_2, bn1_g_2, bn1_b_2,
           W2_2, b2_2, bn_g_2, bn_b_2, ln_g_2, ln_b_2):
    bn_scale = 1.0 / jnp.sqrt(1.0 + 1e-5)
    src_p = jnp.concatenate(
        [edge_index[0], jnp.zeros((E_PAD - E,), jnp.int32)]
    ).reshape(NC * NS, CHUNKS_PER_TILE, CHUNK)
    dst_p = jnp.concatenate(
        [edge_index[1], jnp.full((E_PAD - E,), N, jnp.int32)]
    ).reshape(NC * NS, CHUNKS_PER_TILE, CHUNK)
    sd_t = jnp.stack([src_p, dst_p], axis=2)
    ea_pad = jnp.concatenate(
        [edge_attr, jnp.zeros((E_PAD - E, DE), jnp.float32)])
    zeros = jnp.zeros((N_PAD, H), jnp.float32)

    Ws, Es = [], []
    for (W_e, b_e, eps, W1, b1, g1, bb1, W2, b2, g2, bb2, lg, lb) in (
        (W_edge_0, b_edge_0, eps_0, W1_0, b1_0, bn1_g_0, bn1_b_0, W2_0, b2_0,
         bn_g_0, bn_b_0, ln_g_0, ln_b_0),
        (W_edge_1, b_edge_1, eps_1, W1_1, b1_1, bn1_g_1, bn1_b_1, W2_1, b2_1,
         bn_g_1, bn_b_1, ln_g_1, ln_b_1),
        (W_edge_2, b_edge_2, eps_2, W1_2, b1_2, bn1_g_2, bn1_b_2, W2_2, b2_2,
         bn_g_2, bn_b_2, ln_g_2, ln_b_2),
    ):
        s1 = bn_scale * g1
        w1f = W1 * s1[None, :]
        b1f = (b1 * s1 + bb1)[None, :]
        s2 = bn_scale * g2
        w2f = W2 * s2[None, :]
        b2f = (b2 * s2 + bb2)[None, :]
        Ws.append((eps.reshape(1), w1f, b1f, w2f, b2f,
                   lg[None, :], lb[None, :]))
        Es.append((W_e, b_e))

    w_cat = jnp.concatenate([Es[0][0], Es[1][0], Es[2][0]], axis=1)
    b_cat = jnp.concatenate([Es[0][1], Es[1][1], Es[2][1]])[None, :]
    e0, e1, e2 = _edge_mlp(ea_pad, w_cat, b_cat)

    h = x
    for i, e in enumerate((e0, e1, e2)):
        eps, w1f, b1f, w2f, b2f, lg, lb = Ws[i]
        part = _sc_message_pass(h, e, sd_t, zeros)[:, :N]
        h = _node_mlp(h, part, w1f, b1f, w2f, b2f, lg, lb, eps,
                      residual=(i == 1))
    return h


# speedup vs baseline: 1.0623x; 1.0124x over previous
import functools

import jax
import jax.numpy as jnp
from jax import lax
from jax.experimental import pallas as pl
from jax.experimental.pallas import tpu as pltpu
from jax.experimental.pallas import tpu_sc as plsc

N = 10000
E = 320000
D = 128
DE = 16
H = 128

NC = 2
NS = 16
CHUNK = 128
CHUNKS_PER_TILE = 80
E_PAD = NC * NS * CHUNKS_PER_TILE * CHUNK
N_PAD = 10112
ROWS_PER_TILE = N_PAD // NS
NIB = 4


def _sc_message_pass_body(h_hbm, e_hbm, sd_hbm, zeros_hbm,
                          out_hbm, ib, rows_v, e_v, agg_sh,
                          gsem, esem, isem):
    c = lax.axis_index("c")
    s = lax.axis_index("s")
    wid = c * NS + s
    base = s * ROWS_PER_TILE

    pltpu.sync_copy(zeros_hbm.at[pl.ds(base, ROWS_PER_TILE)],
                    agg_sh.at[pl.ds(base, ROWS_PER_TILE)])
    plsc.subcore_barrier()

    def fetch_idx_async(k, j):
        pltpu.async_copy(sd_hbm.at[wid, k], ib[j], isem[j])

    def fetch_gather(ij, b):
        pltpu.async_copy(h_hbm.at[ib[ij].at[0]], rows_v[b], gsem[b])

    def fetch_e(k):
        eoff = (wid * CHUNKS_PER_TILE + k) * CHUNK
        pltpu.async_copy(e_hbm.at[pl.ds(eoff, CHUNK)], e_v, esem)

    pltpu.sync_copy(sd_hbm.at[wid, 0], ib[0])
    fetch_idx_async(1, 1)
    fetch_gather(0, 0)
    fetch_e(0)

    def step(k, j):
        b = j % 2
        nb = (b + 1) % 2
        pltpu.make_async_copy(h_hbm.at[ib[0].at[0]], rows_v[b],
                              gsem[b]).wait()
        pltpu.make_async_copy(e_hbm.at[pl.ds(0, CHUNK)], e_v, esem).wait()

        def row_body(r, carry2):
            for rr in range(2):
                for jj in range(H // 16):
                    sl = pl.ds(jj * 16, 16)
                    rows_v[b][2 * r + rr, sl] = jnp.maximum(
                        rows_v[b][2 * r + rr, sl] + e_v[2 * r + rr, sl], 0.0)
            return carry2

        lax.fori_loop(0, CHUNK // 2, row_body, 0, unroll=False)

        @pl.when(k + 1 < CHUNKS_PER_TILE)
        def _():
            @pl.when(k + 2 < CHUNKS_PER_TILE)
            def _():
                fetch_idx_async(k + 2, (j + 2) % NIB)

            fetch_e(k + 1)
            pltpu.make_async_copy(sd_hbm.at[wid, 0], ib[(j + 1) % NIB],
                                  isem[(j + 1) % NIB]).wait()
            fetch_gather((j + 1) % NIB, nb)

        pltpu.sync_copy(rows_v[b], agg_sh.at[ib[j].at[1]], add=True)
        return None

    def outer_body(kk, carry):
        for j in range(NIB):
            step(kk * NIB + j, j)
        return carry

    lax.fori_loop(0, CHUNKS_PER_TILE // NIB, outer_body, 0, unroll=False)

    plsc.subcore_barrier()
    pltpu.sync_copy(agg_sh.at[pl.ds(base, ROWS_PER_TILE)],
                    out_hbm.at[c, pl.ds(base, ROWS_PER_TILE)])


def _sc_message_pass(h, e, sd_t, zeros):
    mesh = plsc.VectorSubcoreMesh(core_axis_name="c", subcore_axis_name="s")
    fn = pl.kernel(
        _sc_message_pass_body,
        out_type=jax.ShapeDtypeStruct((NC, N_PAD, H), jnp.float32),
        mesh=mesh,
        scratch_types=[
            [pltpu.VMEM((2, CHUNK), jnp.int32)] * NIB,
            [pltpu.VMEM((CHUNK, H), jnp.float32)] * 2,
            pltpu.VMEM((CHUNK, H), jnp.float32),
            pltpu.VMEM_SHARED((N_PAD, H), jnp.float32),
            [pltpu.SemaphoreType.DMA] * 2,
            pltpu.SemaphoreType.DMA,
            [pltpu.SemaphoreType.DMA] * NIB,
        ],
    )
    return fn(h, e, sd_t, zeros)


def _edge_mlp_body(ea_ref, w_ref, b_ref, o0_ref, o1_ref, o2_ref):
    v = jnp.dot(ea_ref[...], w_ref[...],
                preferred_element_type=jnp.float32) + b_ref[...]
    o0_ref[...] = v[:, :H]
    o1_ref[...] = v[:, H:2 * H]
    o2_ref[...] = v[:, 2 * H:]


def _edge_mlp(edge_attr, w_cat, b_cat):
    BE = 4096
    grid = (E_PAD // BE,)
    out = jax.ShapeDtypeStruct((E_PAD, H), jnp.float32)
    return pl.pallas_call(
        _edge_mlp_body,
        grid=grid,
        in_specs=[
            pl.BlockSpec((BE, DE), lambda i: (i, 0)),
            pl.BlockSpec((DE, 3 * H), lambda i: (0, 0)),
            pl.BlockSpec((1, 3 * H), lambda i: (0, 0)),
        ],
        out_specs=[pl.BlockSpec((BE, H), lambda i: (i, 0))] * 3,
        out_shape=[out] * 3,
    )(edge_attr, w_cat, b_cat)


def _node_mlp_body(h_ref, part_ref, w1_ref, b1_ref, w2_ref, b2_ref,
                   lng_ref, lnb_ref, eps_ref, o_ref, *, residual):
    h = h_ref[...]
    agg = part_ref[0] + part_ref[1]
    z = (1.0 + eps_ref[0]) * h + agg
    z1 = jnp.dot(z, w1_ref[...], preferred_element_type=jnp.float32)
    z1 = jnp.maximum(z1 + b1_ref[...], 0.0)
    z2 = jnp.dot(z1, w2_ref[...], preferred_element_type=jnp.float32)
    z2 = z2 + b2_ref[...]
    mu = jnp.mean(z2, axis=-1, keepdims=True)
    var = jnp.mean((z2 - mu) ** 2, axis=-1, keepdims=True)
    zn = (z2 - mu) * lax.rsqrt(var + 1e-5) * lng_ref[...] + lnb_ref[...]
    zr = jnp.maximum(zn, 0.0)
    if residual:
        o_ref[...] = h + 0.3 * zr
    else:
        o_ref[...] = zr


def _node_mlp(h, part, w1, b1, w2, b2, lng, lnb, eps, residual):
    BN = 1000
    grid = (N // BN,)
    body = functools.partial(_node_mlp_body, residual=residual)
    return pl.pallas_call(
        body,
        grid=grid,
        in_specs=[
            pl.BlockSpec((BN, H), lambda i: (i, 0)),
            pl.BlockSpec((NC, BN, H), lambda i: (0, i, 0)),
            pl.BlockSpec((H, 2 * H), lambda i: (0, 0)),
            pl.BlockSpec((1, 2 * H), lambda i: (0, 0)),
            pl.BlockSpec((2 * H, H), lambda i: (0, 0)),
            pl.BlockSpec((1, H), lambda i: (0, 0)),
            pl.BlockSpec((1, H), lambda i: (0, 0)),
            pl.BlockSpec((1, H), lambda i: (0, 0)),
            pl.BlockSpec(memory_space=pltpu.SMEM),
        ],
        out_specs=pl.BlockSpec((BN, H), lambda i: (i, 0)),
        out_shape=jax.ShapeDtypeStruct((N, H), jnp.float32),
    )(h, part, w1, b1, w2, b2, lng, lnb, eps)


def kernel(x, edge_index, edge_attr,
           W_edge_0, b_edge_0, eps_0, W1_0, b1_0, bn1_g_0, bn1_b_0,
           W2_0, b2_0, bn_g_0, bn_b_0, ln_g_0, ln_b_0,
           W_edge_1, b_edge_1, eps_1, W1_1, b1_1, bn1_g_1, bn1_b_1,
           W2_1, b2_1, bn_g_1, bn_b_1, ln_g_1, ln_b_1,
           W_edge_2, b_edge_2, eps_2, W1_2, b1_2, bn1_g_2, bn1_b_2,
           W2_2, b2_2, bn_g_2, bn_b_2, ln_g_2, ln_b_2):
    bn_scale = 1.0 / jnp.sqrt(1.0 + 1e-5)
    src_p = jnp.concatenate(
        [edge_index[0], jnp.zeros((E_PAD - E,), jnp.int32)]
    ).reshape(NC * NS, CHUNKS_PER_TILE, CHUNK)
    dst_p = jnp.concatenate(
        [edge_index[1], jnp.full((E_PAD - E,), N, jnp.int32)]
    ).reshape(NC * NS, CHUNKS_PER_TILE, CHUNK)
    sd_t = jnp.stack([src_p, dst_p], axis=2)
    ea_pad = jnp.concatenate(
        [edge_attr, jnp.zeros((E_PAD - E, DE), jnp.float32)])
    zeros = jnp.zeros((N_PAD, H), jnp.float32)

    Ws, Es = [], []
    for (W_e, b_e, eps, W1, b1, g1, bb1, W2, b2, g2, bb2, lg, lb) in (
        (W_edge_0, b_edge_0, eps_0, W1_0, b1_0, bn1_g_0, bn1_b_0, W2_0, b2_0,
         bn_g_0, bn_b_0, ln_g_0, ln_b_0),
        (W_edge_1, b_edge_1, eps_1, W1_1, b1_1, bn1_g_1, bn1_b_1, W2_1, b2_1,
         bn_g_1, bn_b_1, ln_g_1, ln_b_1),
        (W_edge_2, b_edge_2, eps_2, W1_2, b1_2, bn1_g_2, bn1_b_2, W2_2, b2_2,
         bn_g_2, bn_b_2, ln_g_2, ln_b_2),
    ):
        s1 = bn_scale * g1
        w1f = W1 * s1[None, :]
        b1f = (b1 * s1 + bb1)[None, :]
        s2 = bn_scale * g2
        w2f = W2 * s2[None, :]
        b2f = (b2 * s2 + bb2)[None, :]
        Ws.append((eps.reshape(1), w1f, b1f, w2f, b2f,
                   lg[None, :], lb[None, :]))
        Es.append((W_e, b_e))

    w_cat = jnp.concatenate([Es[0][0], Es[1][0], Es[2][0]], axis=1)
    b_cat = jnp.concatenate([Es[0][1], Es[1][1], Es[2][1]])[None, :]
    e0, e1, e2 = _edge_mlp(ea_pad, w_cat, b_cat)

    h = x
    for i, e in enumerate((e0, e1, e2)):
        eps, w1f, b1f, w2f, b2f, lg, lb = Ws[i]
        part = _sc_message_pass(h, e, sd_t, zeros)[:, :N]
        h = _node_mlp(h, part, w1f, b1f, w2f, b2f, lg, lb, eps,
                      residual=(i == 1))
    return h


# speedup vs baseline: 1.9369x; 1.8233x over previous
import functools

import jax
import jax.numpy as jnp
from jax import lax
from jax.experimental import pallas as pl
from jax.experimental.pallas import tpu as pltpu
from jax.experimental.pallas import tpu_sc as plsc

N = 10000
E = 320000
D = 128
DE = 16
H = 128

NC = 2
NS = 16
CHUNK = 128
CHUNKS_PER_TILE = 80
E_PAD = NC * NS * CHUNKS_PER_TILE * CHUNK
N_PAD = 10112
ROWS_PER_TILE = N_PAD // NS
NIB = 4


def _sc_message_pass_body(h_hbm, e_hbm, sd_hbm, zeros_hbm,
                          out_hbm, ib, rows_v, e_v, agg_sh,
                          gsem, esem, isem):
    c = lax.axis_index("c")
    s = lax.axis_index("s")
    wid = c * NS + s
    base = s * ROWS_PER_TILE

    pltpu.sync_copy(zeros_hbm.at[pl.ds(base, ROWS_PER_TILE)],
                    agg_sh.at[pl.ds(base, ROWS_PER_TILE)])
    plsc.subcore_barrier()

    def fetch_idx_async(k, j):
        pltpu.async_copy(sd_hbm.at[wid, k], ib[j], isem[j])

    def fetch_gather(ij, b):
        pltpu.async_copy(h_hbm.at[ib[ij].at[0]], rows_v[b], gsem[b])

    def fetch_e(k):
        eoff = (wid * CHUNKS_PER_TILE + k) * CHUNK
        pltpu.async_copy(e_hbm.at[pl.ds(eoff, CHUNK)], e_v, esem)

    pltpu.sync_copy(sd_hbm.at[wid, 0], ib[0])
    fetch_idx_async(1, 1)
    fetch_gather(0, 0)
    fetch_e(0)

    def step(k, j):
        b = j % 2
        nb = (b + 1) % 2
        pltpu.make_async_copy(h_hbm.at[ib[0].at[0]], rows_v[b],
                              gsem[b]).wait()
        pltpu.make_async_copy(e_hbm.at[pl.ds(0, CHUNK)], e_v, esem).wait()

        def row_body(r, carry2):
            for rr in range(2):
                for jj in range(H // 16):
                    sl = pl.ds(jj * 16, 16)
                    rows_v[b][2 * r + rr, sl] = jnp.maximum(
                        rows_v[b][2 * r + rr, sl] + e_v[2 * r + rr, sl], 0.0)
            return carry2

        lax.fori_loop(0, CHUNK // 2, row_body, 0, unroll=False)

        @pl.when(k + 1 < CHUNKS_PER_TILE)
        def _():
            @pl.when(k + 2 < CHUNKS_PER_TILE)
            def _():
                fetch_idx_async(k + 2, (j + 2) % NIB)

            fetch_e(k + 1)
            pltpu.make_async_copy(sd_hbm.at[wid, 0], ib[(j + 1) % NIB],
                                  isem[(j + 1) % NIB]).wait()
            fetch_gather((j + 1) % NIB, nb)

        pltpu.sync_copy(rows_v[b], agg_sh.at[ib[j].at[1]], add=True)
        return None

    def outer_body(kk, carry):
        for j in range(NIB):
            step(kk * NIB + j, j)
        return carry

    lax.fori_loop(0, CHUNKS_PER_TILE // NIB, outer_body, 0, unroll=False)

    plsc.subcore_barrier()
    pltpu.sync_copy(agg_sh.at[pl.ds(base, ROWS_PER_TILE)],
                    out_hbm.at[c, pl.ds(base, ROWS_PER_TILE)])


def _sc_message_pass(h, e, sd_t, zeros):
    mesh = plsc.VectorSubcoreMesh(core_axis_name="c", subcore_axis_name="s")
    fn = pl.kernel(
        _sc_message_pass_body,
        out_type=jax.ShapeDtypeStruct((NC, N_PAD, H), jnp.float32),
        mesh=mesh,
        scratch_types=[
            [pltpu.VMEM((2, CHUNK), jnp.int32)] * NIB,
            [pltpu.VMEM((CHUNK, H), jnp.float32)] * 2,
            pltpu.VMEM((CHUNK, H), jnp.float32),
            pltpu.VMEM_SHARED((N_PAD, H), jnp.float32),
            [pltpu.SemaphoreType.DMA] * 2,
            pltpu.SemaphoreType.DMA,
            [pltpu.SemaphoreType.DMA] * NIB,
        ],
    )
    return fn(h, e, sd_t, zeros)


def _edge_mlp_body(ea_ref, w_ref, b_ref, o0_ref, o1_ref, o2_ref):
    v = jnp.dot(ea_ref[...], w_ref[...],
                preferred_element_type=jnp.float32) + b_ref[...]
    o0_ref[...] = v[:, :H]
    o1_ref[...] = v[:, H:2 * H]
    o2_ref[...] = v[:, 2 * H:]


def _edge_mlp(edge_attr, w_cat, b_cat):
    BE = 4096
    grid = (E_PAD // BE,)
    out = jax.ShapeDtypeStruct((E_PAD, H), jnp.float32)
    return pl.pallas_call(
        _edge_mlp_body,
        grid=grid,
        in_specs=[
            pl.BlockSpec((BE, DE), lambda i: (i, 0)),
            pl.BlockSpec((DE, 3 * H), lambda i: (0, 0)),
            pl.BlockSpec((1, 3 * H), lambda i: (0, 0)),
        ],
        out_specs=[pl.BlockSpec((BE, H), lambda i: (i, 0))] * 3,
        out_shape=[out] * 3,
    )(edge_attr, w_cat, b_cat)


def _node_mlp_body(h_ref, part_ref, w1_ref, b1_ref, w2_ref, b2_ref,
                   lng_ref, lnb_ref, eps_ref, o_ref, *, residual):
    h = h_ref[...]
    agg = part_ref[0] + part_ref[1]
    z = (1.0 + eps_ref[0]) * h + agg
    z1 = jnp.dot(z, w1_ref[...], preferred_element_type=jnp.float32)
    z1 = jnp.maximum(z1 + b1_ref[...], 0.0)
    z2 = jnp.dot(z1, w2_ref[...], preferred_element_type=jnp.float32)
    z2 = z2 + b2_ref[...]
    mu = jnp.mean(z2, axis=-1, keepdims=True)
    var = jnp.mean((z2 - mu) ** 2, axis=-1, keepdims=True)
    zn = (z2 - mu) * lax.rsqrt(var + 1e-5) * lng_ref[...] + lnb_ref[...]
    zr = jnp.maximum(zn, 0.0)
    if residual:
        o_ref[...] = h + 0.3 * zr
    else:
        o_ref[...] = zr


def _node_mlp(h, part, w1, b1, w2, b2, lng, lnb, eps, residual):
    BN = 1000
    grid = (N // BN,)
    body = functools.partial(_node_mlp_body, residual=residual)
    return pl.pallas_call(
        body,
        grid=grid,
        in_specs=[
            pl.BlockSpec((BN, H), lambda i: (i, 0)),
            pl.BlockSpec((NC, BN, H), lambda i: (0, i, 0)),
            pl.BlockSpec((H, 2 * H), lambda i: (0, 0)),
            pl.BlockSpec((1, 2 * H), lambda i: (0, 0)),
            pl.BlockSpec((2 * H, H), lambda i: (0, 0)),
            pl.BlockSpec((1, H), lambda i: (0, 0)),
            pl.BlockSpec((1, H), lambda i: (0, 0)),
            pl.BlockSpec((1, H), lambda i: (0, 0)),
            pl.BlockSpec(memory_space=pltpu.SMEM),
        ],
        out_specs=pl.BlockSpec((BN, H), lambda i: (i, 0)),
        out_shape=jax.ShapeDtypeStruct((N, H), jnp.float32),
    )(h, part, w1, b1, w2, b2, lng, lnb, eps)


def kernel(x, edge_index, edge_attr,
           W_edge_0, b_edge_0, eps_0, W1_0, b1_0, bn1_g_0, bn1_b_0,
           W2_0, b2_0, bn_g_0, bn_b_0, ln_g_0, ln_b_0,
           W_edge_1, b_edge_1, eps_1, W1_1, b1_1, bn1_g_1, bn1_b_1,
           W2_1, b2_1, bn_g_1, bn_b_1, ln_g_1, ln_b_1,
           W_edge_2, b_edge_2, eps_2, W1_2, b1_2, bn1_g_2, bn1_b_2,
           W2_2, b2_2, bn_g_2, bn_b_2, ln_g_2, ln_b_2):
    bn_scale = 1.0 / jnp.sqrt(1.0 + 1e-5)
    pad_i = jnp.arange(E_PAD - E, dtype=jnp.int32)
    src_p = jnp.concatenate(
        [edge_index[0], pad_i % N]
    ).reshape(NC * NS, CHUNKS_PER_TILE, CHUNK)
    dst_p = jnp.concatenate(
        [edge_index[1], N + pad_i % (N_PAD - N)]
    ).reshape(NC * NS, CHUNKS_PER_TILE, CHUNK)
    sd_t = jnp.stack([src_p, dst_p], axis=2)
    ea_pad = jnp.concatenate(
        [edge_attr, jnp.zeros((E_PAD - E, DE), jnp.float32)])
    zeros = jnp.zeros((N_PAD, H), jnp.float32)

    Ws, Es = [], []
    for (W_e, b_e, eps, W1, b1, g1, bb1, W2, b2, g2, bb2, lg, lb) in (
        (W_edge_0, b_edge_0, eps_0, W1_0, b1_0, bn1_g_0, bn1_b_0, W2_0, b2_0,
         bn_g_0, bn_b_0, ln_g_0, ln_b_0),
        (W_edge_1, b_edge_1, eps_1, W1_1, b1_1, bn1_g_1, bn1_b_1, W2_1, b2_1,
         bn_g_1, bn_b_1, ln_g_1, ln_b_1),
        (W_edge_2, b_edge_2, eps_2, W1_2, b1_2, bn1_g_2, bn1_b_2, W2_2, b2_2,
         bn_g_2, bn_b_2, ln_g_2, ln_b_2),
    ):
        s1 = bn_scale * g1
        w1f = W1 * s1[None, :]
        b1f = (b1 * s1 + bb1)[None, :]
        s2 = bn_scale * g2
        w2f = W2 * s2[None, :]
        b2f = (b2 * s2 + bb2)[None, :]
        Ws.append((eps.reshape(1), w1f, b1f, w2f, b2f,
                   lg[None, :], lb[None, :]))
        Es.append((W_e, b_e))

    w_cat = jnp.concatenate([Es[0][0], Es[1][0], Es[2][0]], axis=1)
    b_cat = jnp.concatenate([Es[0][1], Es[1][1], Es[2][1]])[None, :]
    e0, e1, e2 = _edge_mlp(ea_pad, w_cat, b_cat)

    h = x
    for i, e in enumerate((e0, e1, e2)):
        eps, w1f, b1f, w2f, b2f, lg, lb = Ws[i]
        part = _sc_message_pass(h, e, sd_t, zeros)[:, :N]
        h = _node_mlp(h, part, w1f, b1f, w2f, b2f, lg, lb, eps,
                      residual=(i == 1))
    return h


# speedup vs baseline: 1.9885x; 1.0266x over previous
import functools

import jax
import jax.numpy as jnp
from jax import lax
from jax.experimental import pallas as pl
from jax.experimental.pallas import tpu as pltpu
from jax.experimental.pallas import tpu_sc as plsc

N = 10000
E = 320000
D = 128
DE = 16
H = 128

NC = 2
NS = 16
CHUNK = 128
CHUNKS_PER_TILE = 80
E_PAD = NC * NS * CHUNKS_PER_TILE * CHUNK
N_PAD = 10112
ROWS_PER_TILE = N_PAD // NS
NIB = 4


def _sc_message_pass_body(h_hbm, e_hbm, sd_hbm, zeros_hbm,
                          out_hbm, ib, rows_v, e_v, agg_sh,
                          gsem, esem, isem):
    c = lax.axis_index("c")
    s = lax.axis_index("s")
    wid = c * NS + s
    base = s * ROWS_PER_TILE

    pltpu.sync_copy(zeros_hbm.at[pl.ds(base, ROWS_PER_TILE)],
                    agg_sh.at[pl.ds(base, ROWS_PER_TILE)])
    plsc.subcore_barrier()

    def fetch_idx_async(k, j):
        pltpu.async_copy(sd_hbm.at[wid, k], ib[j], isem[j])

    def fetch_gather(ij, b):
        pltpu.async_copy(h_hbm.at[ib[ij].at[0]], rows_v[b], gsem[b])

    def fetch_e(k):
        eoff = (wid * CHUNKS_PER_TILE + k) * CHUNK
        pltpu.async_copy(e_hbm.at[pl.ds(eoff, CHUNK)], e_v, esem)

    pltpu.sync_copy(sd_hbm.at[wid, 0], ib[0])
    fetch_idx_async(1, 1)
    fetch_gather(0, 0)
    fetch_e(0)

    def step(k, j):
        b = j % 2
        nb = (b + 1) % 2
        pltpu.make_async_copy(h_hbm.at[ib[0].at[0]], rows_v[b],
                              gsem[b]).wait()
        pltpu.make_async_copy(e_hbm.at[pl.ds(0, CHUNK)], e_v, esem).wait()

        def row_body(r, carry2):
            for rr in range(2):
                for jj in range(H // 16):
                    sl = pl.ds(jj * 16, 16)
                    rows_v[b][2 * r + rr, sl] = jnp.maximum(
                        rows_v[b][2 * r + rr, sl] + e_v[2 * r + rr, sl], 0.0)
            return carry2

        lax.fori_loop(0, CHUNK // 2, row_body, 0, unroll=False)

        @pl.when(k + 1 < CHUNKS_PER_TILE)
        def _():
            @pl.when(k + 2 < CHUNKS_PER_TILE)
            def _():
                fetch_idx_async(k + 2, (j + 2) % NIB)

            fetch_e(k + 1)
            pltpu.make_async_copy(sd_hbm.at[wid, 0], ib[(j + 1) % NIB],
                                  isem[(j + 1) % NIB]).wait()
            fetch_gather((j + 1) % NIB, nb)

        pltpu.sync_copy(rows_v[b], agg_sh.at[ib[j].at[1]], add=True)
        return None

    def outer_body(kk, carry):
        for j in range(NIB):
            step(kk * NIB + j, j)
        return carry

    lax.fori_loop(0, CHUNKS_PER_TILE // NIB, outer_body, 0, unroll=False)

    plsc.subcore_barrier()
    pltpu.sync_copy(agg_sh.at[pl.ds(base, ROWS_PER_TILE)],
                    out_hbm.at[c, pl.ds(base, ROWS_PER_TILE)])


def _sc_message_pass(h, e, sd_t, zeros):
    mesh = plsc.VectorSubcoreMesh(core_axis_name="c", subcore_axis_name="s")
    fn = pl.kernel(
        _sc_message_pass_body,
        out_type=jax.ShapeDtypeStruct((NC, N_PAD, H), jnp.float32),
        mesh=mesh,
        scratch_types=[
            [pltpu.VMEM((2, CHUNK), jnp.int32)] * NIB,
            [pltpu.VMEM((CHUNK, H), jnp.float32)] * 2,
            pltpu.VMEM((CHUNK, H), jnp.float32),
            pltpu.VMEM_SHARED((N_PAD, H), jnp.float32),
            [pltpu.SemaphoreType.DMA] * 2,
            pltpu.SemaphoreType.DMA,
            [pltpu.SemaphoreType.DMA] * NIB,
        ],
    )
    return fn(h, e, sd_t, zeros)


def _edge_mlp_body(ea_ref, w_ref, b_ref, o_ref):
    o_ref[...] = jnp.dot(ea_ref[...], w_ref[...],
                         preferred_element_type=jnp.float32) + b_ref[...]


def _edge_mlp(edge_attr, w_e, b_e):
    BE = 4096
    grid = (E_PAD // BE,)
    return pl.pallas_call(
        _edge_mlp_body,
        grid=grid,
        in_specs=[
            pl.BlockSpec((BE, DE), lambda i: (i, 0)),
            pl.BlockSpec((DE, H), lambda i: (0, 0)),
            pl.BlockSpec((1, H), lambda i: (0, 0)),
        ],
        out_specs=pl.BlockSpec((BE, H), lambda i: (i, 0)),
        out_shape=jax.ShapeDtypeStruct((E_PAD, H), jnp.float32),
    )(edge_attr, w_e, b_e)


def _node_mlp_body(h_ref, part_ref, w1_ref, b1_ref, w2_ref, b2_ref,
                   lng_ref, lnb_ref, eps_ref, o_ref, *, residual):
    h = h_ref[...]
    agg = part_ref[0] + part_ref[1]
    z = (1.0 + eps_ref[0]) * h + agg
    z1 = jnp.dot(z, w1_ref[...], preferred_element_type=jnp.float32)
    z1 = jnp.maximum(z1 + b1_ref[...], 0.0)
    z2 = jnp.dot(z1, w2_ref[...], preferred_element_type=jnp.float32)
    z2 = z2 + b2_ref[...]
    mu = jnp.mean(z2, axis=-1, keepdims=True)
    var = jnp.mean((z2 - mu) ** 2, axis=-1, keepdims=True)
    zn = (z2 - mu) * lax.rsqrt(var + 1e-5) * lng_ref[...] + lnb_ref[...]
    zr = jnp.maximum(zn, 0.0)
    if residual:
        o_ref[...] = h + 0.3 * zr
    else:
        o_ref[...] = zr


def _node_mlp(h, part, w1, b1, w2, b2, lng, lnb, eps, residual):
    BN = 1000
    grid = (N // BN,)
    body = functools.partial(_node_mlp_body, residual=residual)
    return pl.pallas_call(
        body,
        grid=grid,
        in_specs=[
            pl.BlockSpec((BN, H), lambda i: (i, 0)),
            pl.BlockSpec((NC, BN, H), lambda i: (0, i, 0)),
            pl.BlockSpec((H, 2 * H), lambda i: (0, 0)),
            pl.BlockSpec((1, 2 * H), lambda i: (0, 0)),
            pl.BlockSpec((2 * H, H), lambda i: (0, 0)),
            pl.BlockSpec((1, H), lambda i: (0, 0)),
            pl.BlockSpec((1, H), lambda i: (0, 0)),
            pl.BlockSpec((1, H), lambda i: (0, 0)),
            pl.BlockSpec(memory_space=pltpu.SMEM),
        ],
        out_specs=pl.BlockSpec((BN, H), lambda i: (i, 0)),
        out_shape=jax.ShapeDtypeStruct((N, H), jnp.float32),
    )(h, part, w1, b1, w2, b2, lng, lnb, eps)


def kernel(x, edge_index, edge_attr,
           W_edge_0, b_edge_0, eps_0, W1_0, b1_0, bn1_g_0, bn1_b_0,
           W2_0, b2_0, bn_g_0, bn_b_0, ln_g_0, ln_b_0,
           W_edge_1, b_edge_1, eps_1, W1_1, b1_1, bn1_g_1, bn1_b_1,
           W2_1, b2_1, bn_g_1, bn_b_1, ln_g_1, ln_b_1,
           W_edge_2, b_edge_2, eps_2, W1_2, b1_2, bn1_g_2, bn1_b_2,
           W2_2, b2_2, bn_g_2, bn_b_2, ln_g_2, ln_b_2):
    bn_scale = 1.0 / jnp.sqrt(1.0 + 1e-5)
    pad_i = jnp.arange(E_PAD - E, dtype=jnp.int32)
    src_p = jnp.concatenate(
        [edge_index[0], pad_i % N]
    ).reshape(NC * NS, CHUNKS_PER_TILE, CHUNK)
    dst_p = jnp.concatenate(
        [edge_index[1], N + pad_i % (N_PAD - N)]
    ).reshape(NC * NS, CHUNKS_PER_TILE, CHUNK)
    sd_t = jnp.stack([src_p, dst_p], axis=2)
    ea_pad = jnp.concatenate(
        [edge_attr, jnp.zeros((E_PAD - E, DE), jnp.float32)])
    zeros = jnp.zeros((N_PAD, H), jnp.float32)

    Ws, Es = [], []
    for (W_e, b_e, eps, W1, b1, g1, bb1, W2, b2, g2, bb2, lg, lb) in (
        (W_edge_0, b_edge_0, eps_0, W1_0, b1_0, bn1_g_0, bn1_b_0, W2_0, b2_0,
         bn_g_0, bn_b_0, ln_g_0, ln_b_0),
        (W_edge_1, b_edge_1, eps_1, W1_1, b1_1, bn1_g_1, bn1_b_1, W2_1, b2_1,
         bn_g_1, bn_b_1, ln_g_1, ln_b_1),
        (W_edge_2, b_edge_2, eps_2, W1_2, b1_2, bn1_g_2, bn1_b_2, W2_2, b2_2,
         bn_g_2, bn_b_2, ln_g_2, ln_b_2),
    ):
        s1 = bn_scale * g1
        w1f = W1 * s1[None, :]
        b1f = (b1 * s1 + bb1)[None, :]
        s2 = bn_scale * g2
        w2f = W2 * s2[None, :]
        b2f = (b2 * s2 + bb2)[None, :]
        Ws.append((eps.reshape(1), w1f, b1f, w2f, b2f,
                   lg[None, :], lb[None, :]))
        Es.append((W_e, b_e))

    h = x
    for i in range(3):
        eps, w1f, b1f, w2f, b2f, lg, lb = Ws[i]
        e = _edge_mlp(ea_pad, Es[i][0], Es[i][1][None, :])
        part = _sc_message_pass(h, e, sd_t, zeros)[:, :N]
        h = _node_mlp(h, part, w1f, b1f, w2f, b2f, lg, lb, eps,
                      residual=(i == 1))
    return h


# speedup vs baseline: 2.0248x; 1.0183x over previous
import functools

import jax
import jax.numpy as jnp
from jax import lax
from jax.experimental import pallas as pl
from jax.experimental.pallas import tpu as pltpu
from jax.experimental.pallas import tpu_sc as plsc

N = 10000
E = 320000
D = 128
DE = 16
H = 128

NC = 2
NS = 16
CHUNK = 128
CHUNKS_PER_TILE = 80
E_PAD = NC * NS * CHUNKS_PER_TILE * CHUNK
N_PAD = 10112
ROWS_PER_TILE = N_PAD // NS
NIB = 4


def _sc_message_pass_body(h_hbm, e_hbm, sd_hbm, zeros_hbm,
                          out_hbm, ib, rows_v, e_v, agg_sh,
                          gsem, esem, isem):
    c = lax.axis_index("c")
    s = lax.axis_index("s")
    wid = c * NS + s
    base = s * ROWS_PER_TILE

    pltpu.sync_copy(zeros_hbm.at[pl.ds(base, ROWS_PER_TILE)],
                    agg_sh.at[pl.ds(base, ROWS_PER_TILE)])
    plsc.subcore_barrier()

    def fetch_idx_async(k, j):
        pltpu.async_copy(sd_hbm.at[wid, k], ib[j], isem[j])

    def fetch_gather(ij, b):
        pltpu.async_copy(h_hbm.at[ib[ij].at[0]], rows_v[b], gsem[b])

    def fetch_e(k):
        eoff = (wid * CHUNKS_PER_TILE + k) * CHUNK
        pltpu.async_copy(e_hbm.at[pl.ds(eoff, CHUNK)], e_v, esem)

    pltpu.sync_copy(sd_hbm.at[wid, 0], ib[0])
    fetch_idx_async(1, 1)
    fetch_gather(0, 0)
    fetch_e(0)

    def step(k, j):
        b = j % 2
        nb = (b + 1) % 2
        pltpu.make_async_copy(h_hbm.at[ib[0].at[0]], rows_v[b],
                              gsem[b]).wait()
        pltpu.make_async_copy(e_hbm.at[pl.ds(0, CHUNK)], e_v, esem).wait()

        def row_body(r, carry2):
            for rr in range(2):
                for jj in range(H // 16):
                    sl = pl.ds(jj * 16, 16)
                    rows_v[b][2 * r + rr, sl] = jnp.maximum(
                        rows_v[b][2 * r + rr, sl] + e_v[2 * r + rr, sl], 0.0)
            return carry2

        lax.fori_loop(0, CHUNK // 2, row_body, 0, unroll=False)

        @pl.when(k + 1 < CHUNKS_PER_TILE)
        def _():
            @pl.when(k + 2 < CHUNKS_PER_TILE)
            def _():
                fetch_idx_async(k + 2, (j + 2) % NIB)

            fetch_e(k + 1)
            pltpu.make_async_copy(sd_hbm.at[wid, 0], ib[(j + 1) % NIB],
                                  isem[(j + 1) % NIB]).wait()
            fetch_gather((j + 1) % NIB, nb)

        pltpu.sync_copy(rows_v[b], agg_sh.at[ib[j].at[1]], add=True)
        return None

    def outer_body(kk, carry):
        for j in range(NIB):
            step(kk * NIB + j, j)
        return carry

    lax.fori_loop(0, CHUNKS_PER_TILE // NIB, outer_body, 0, unroll=False)

    plsc.subcore_barrier()
    pltpu.sync_copy(agg_sh.at[pl.ds(base, ROWS_PER_TILE)],
                    out_hbm.at[c, pl.ds(base, ROWS_PER_TILE)])


def _sc_message_pass(h, e, sd_t, zeros):
    mesh = plsc.VectorSubcoreMesh(core_axis_name="c", subcore_axis_name="s")
    fn = pl.kernel(
        _sc_message_pass_body,
        out_type=jax.ShapeDtypeStruct((NC, N_PAD, H), jnp.float32),
        mesh=mesh,
        scratch_types=[
            [pltpu.VMEM((2, CHUNK), jnp.int32)] * NIB,
            [pltpu.VMEM((CHUNK, H), jnp.float32)] * 2,
            pltpu.VMEM((CHUNK, H), jnp.float32),
            pltpu.VMEM_SHARED((N_PAD, H), jnp.float32),
            [pltpu.SemaphoreType.DMA] * 2,
            pltpu.SemaphoreType.DMA,
            [pltpu.SemaphoreType.DMA] * NIB,
        ],
    )
    return fn(h, e, sd_t, zeros)


def _edge_mlp_body(ea_ref, w_ref, b_ref, o_ref):
    o_ref[...] = jnp.dot(ea_ref[...], w_ref[...],
                         preferred_element_type=jnp.float32) + b_ref[...]


def _edge_mlp(edge_attr, w_e, b_e):
    BE = 4096
    grid = (E_PAD // BE,)
    return pl.pallas_call(
        _edge_mlp_body,
        grid=grid,
        in_specs=[
            pl.BlockSpec((BE, DE), lambda i: (i, 0)),
            pl.BlockSpec((DE, H), lambda i: (0, 0)),
            pl.BlockSpec((1, H), lambda i: (0, 0)),
        ],
        out_specs=pl.BlockSpec((BE, H), lambda i: (i, 0)),
        out_shape=jax.ShapeDtypeStruct((E_PAD, H), jnp.float32),
    )(edge_attr, w_e, b_e)


def _node_mlp_body(h_ref, part_ref, w1_ref, b1_ref, w2_ref, b2_ref,
                   lng_ref, lnb_ref, eps_ref, o_ref, *, residual):
    h = h_ref[...]
    agg = part_ref[0] + part_ref[1]
    z = (1.0 + eps_ref[0]) * h + agg
    z1 = jnp.dot(z, w1_ref[...], preferred_element_type=jnp.float32)
    z1 = jnp.maximum(z1 + b1_ref[...], 0.0)
    z2 = jnp.dot(z1, w2_ref[...], preferred_element_type=jnp.float32)
    z2 = z2 + b2_ref[...]
    mu = jnp.mean(z2, axis=-1, keepdims=True)
    var = jnp.mean((z2 - mu) ** 2, axis=-1, keepdims=True)
    zn = (z2 - mu) * lax.rsqrt(var + 1e-5) * lng_ref[...] + lnb_ref[...]
    zr = jnp.maximum(zn, 0.0)
    if residual:
        o_ref[...] = h + 0.3 * zr
    else:
        o_ref[...] = zr


def _node_mlp(h, part, w1, b1, w2, b2, lng, lnb, eps, residual):
    BN = 1000
    grid = (N // BN,)
    body = functools.partial(_node_mlp_body, residual=residual)
    return pl.pallas_call(
        body,
        grid=grid,
        in_specs=[
            pl.BlockSpec((BN, H), lambda i: (i, 0)),
            pl.BlockSpec((NC, BN, H), lambda i: (0, i, 0)),
            pl.BlockSpec((H, 2 * H), lambda i: (0, 0)),
            pl.BlockSpec((1, 2 * H), lambda i: (0, 0)),
            pl.BlockSpec((2 * H, H), lambda i: (0, 0)),
            pl.BlockSpec((1, H), lambda i: (0, 0)),
            pl.BlockSpec((1, H), lambda i: (0, 0)),
            pl.BlockSpec((1, H), lambda i: (0, 0)),
            pl.BlockSpec(memory_space=pltpu.SMEM),
        ],
        out_specs=pl.BlockSpec((BN, H), lambda i: (i, 0)),
        out_shape=jax.ShapeDtypeStruct((N, H), jnp.float32),
    )(h, part, w1, b1, w2, b2, lng, lnb, eps)


def kernel(x, edge_index, edge_attr,
           W_edge_0, b_edge_0, eps_0, W1_0, b1_0, bn1_g_0, bn1_b_0,
           W2_0, b2_0, bn_g_0, bn_b_0, ln_g_0, ln_b_0,
           W_edge_1, b_edge_1, eps_1, W1_1, b1_1, bn1_g_1, bn1_b_1,
           W2_1, b2_1, bn_g_1, bn_b_1, ln_g_1, ln_b_1,
           W_edge_2, b_edge_2, eps_2, W1_2, b1_2, bn1_g_2, bn1_b_2,
           W2_2, b2_2, bn_g_2, bn_b_2, ln_g_2, ln_b_2):
    bn_scale = 1.0 / jnp.sqrt(1.0 + 1e-5)
    pad_i = jnp.arange(E_PAD - E, dtype=jnp.int32)
    src_p = jnp.concatenate(
        [edge_index[0], pad_i % N]
    ).reshape(NC * NS, CHUNKS_PER_TILE, CHUNK)
    dst_p = jnp.concatenate(
        [edge_index[1], N + pad_i % (N_PAD - N)]
    ).reshape(NC * NS, CHUNKS_PER_TILE, CHUNK)
    sd_t = jnp.stack([src_p, dst_p], axis=2)
    ea_pad = jnp.concatenate(
        [edge_attr, jnp.zeros((E_PAD - E, DE), jnp.float32)])
    zeros = jnp.zeros((N_PAD, H), jnp.float32)

    Ws, Es = [], []
    for (W_e, b_e, eps, W1, b1, g1, bb1, W2, b2, g2, bb2, lg, lb) in (
        (W_edge_0, b_edge_0, eps_0, W1_0, b1_0, bn1_g_0, bn1_b_0, W2_0, b2_0,
         bn_g_0, bn_b_0, ln_g_0, ln_b_0),
        (W_edge_1, b_edge_1, eps_1, W1_1, b1_1, bn1_g_1, bn1_b_1, W2_1, b2_1,
         bn_g_1, bn_b_1, ln_g_1, ln_b_1),
        (W_edge_2, b_edge_2, eps_2, W1_2, b1_2, bn1_g_2, bn1_b_2, W2_2, b2_2,
         bn_g_2, bn_b_2, ln_g_2, ln_b_2),
    ):
        s1 = bn_scale * g1
        w1f = W1 * s1[None, :]
        b1f = (b1 * s1 + bb1)[None, :]
        s2 = bn_scale * g2
        w2f = W2 * s2[None, :]
        b2f = (b2 * s2 + bb2)[None, :]
        Ws.append((eps.reshape(1), w1f, b1f, w2f, b2f,
                   lg[None, :], lb[None, :]))
        Es.append((W_e, b_e))

    h = x
    for i in range(3):
        eps, w1f, b1f, w2f, b2f, lg, lb = Ws[i]
        e = _edge_mlp(ea_pad, Es[i][0], Es[i][1][None, :])
        part = _sc_message_pass(h, e, sd_t, zeros)
        h = _node_mlp(h, part, w1f, b1f, w2f, b2f, lg, lb, eps,
                      residual=(i == 1))
    return h


# speedup vs baseline: 2.0803x; 1.0274x over previous
import functools

import jax
import jax.numpy as jnp
from jax import lax
from jax.experimental import pallas as pl
from jax.experimental.pallas import tpu as pltpu
from jax.experimental.pallas import tpu_sc as plsc

N = 10000
E = 320000
D = 128
DE = 16
H = 128

NC = 2
NS = 16
CHUNK = 128
CHUNKS_PER_TILE = 80
E_PAD = NC * NS * CHUNKS_PER_TILE * CHUNK
N_PAD = 10112
ROWS_PER_TILE = N_PAD // NS
NIB = 4


def _sc_message_pass_body(h_hbm, e_hbm, sd_hbm, zeros_hbm,
                          out_hbm, ib, rows_v, e_v, agg_sh,
                          gsem, esem, isem):
    c = lax.axis_index("c")
    s = lax.axis_index("s")
    wid = c * NS + s
    base = s * ROWS_PER_TILE

    pltpu.sync_copy(zeros_hbm.at[pl.ds(base, ROWS_PER_TILE)],
                    agg_sh.at[pl.ds(base, ROWS_PER_TILE)])
    plsc.subcore_barrier()

    def fetch_idx_async(k, j):
        pltpu.async_copy(sd_hbm.at[wid, k], ib[j], isem[j])

    def fetch_gather(ij, b):
        pltpu.async_copy(h_hbm.at[ib[ij].at[0]], rows_v[b], gsem[b])

    def fetch_e(k):
        eoff = (wid * CHUNKS_PER_TILE + k) * CHUNK
        pltpu.async_copy(e_hbm.at[pl.ds(eoff, CHUNK)], e_v, esem)

    pltpu.sync_copy(sd_hbm.at[wid, 0], ib[0])
    fetch_idx_async(1, 1)
    fetch_gather(0, 0)
    fetch_e(0)

    def step(k, j):
        b = j % 2
        nb = (b + 1) % 2
        pltpu.make_async_copy(h_hbm.at[ib[0].at[0]], rows_v[b],
                              gsem[b]).wait()
        pltpu.make_async_copy(e_hbm.at[pl.ds(0, CHUNK)], e_v, esem).wait()

        def row_body(r, carry2):
            for rr in range(2):
                for jj in range(H // 16):
                    sl = pl.ds(jj * 16, 16)
                    rows_v[b][2 * r + rr, sl] = jnp.maximum(
                        rows_v[b][2 * r + rr, sl] + e_v[2 * r + rr, sl], 0.0)
            return carry2

        lax.fori_loop(0, CHUNK // 2, row_body, 0, unroll=False)

        @pl.when(k + 1 < CHUNKS_PER_TILE)
        def _():
            @pl.when(k + 2 < CHUNKS_PER_TILE)
            def _():
                fetch_idx_async(k + 2, (j + 2) % NIB)

            fetch_e(k + 1)
            pltpu.make_async_copy(sd_hbm.at[wid, 0], ib[(j + 1) % NIB],
                                  isem[(j + 1) % NIB]).wait()
            fetch_gather((j + 1) % NIB, nb)

        pltpu.sync_copy(rows_v[b], agg_sh.at[ib[j].at[1]], add=True)
        return None

    def outer_body(kk, carry):
        for j in range(NIB):
            step(kk * NIB + j, j)
        return carry

    lax.fori_loop(0, CHUNKS_PER_TILE // NIB, outer_body, 0, unroll=False)

    plsc.subcore_barrier()
    pltpu.sync_copy(agg_sh.at[pl.ds(base, ROWS_PER_TILE)],
                    out_hbm.at[c, pl.ds(base, ROWS_PER_TILE)])


def _sc_message_pass(h, e, sd_t, zeros):
    mesh = plsc.VectorSubcoreMesh(core_axis_name="c", subcore_axis_name="s")
    fn = pl.kernel(
        _sc_message_pass_body,
        out_type=jax.ShapeDtypeStruct((NC, N_PAD, H), jnp.float32),
        mesh=mesh,
        scratch_types=[
            [pltpu.VMEM((2, CHUNK), jnp.int32)] * NIB,
            [pltpu.VMEM((CHUNK, H), jnp.float32)] * 2,
            pltpu.VMEM((CHUNK, H), jnp.float32),
            pltpu.VMEM_SHARED((N_PAD, H), jnp.float32),
            [pltpu.SemaphoreType.DMA] * 2,
            pltpu.SemaphoreType.DMA,
            [pltpu.SemaphoreType.DMA] * NIB,
        ],
    )
    return fn(h, e, sd_t, zeros)


def _edge_mlp_body(ea_ref, w_ref, b_ref, o_ref):
    o_ref[...] = jnp.dot(ea_ref[...], w_ref[...],
                         preferred_element_type=jnp.float32) + b_ref[...]


def _edge_mlp(edge_attr, w_e, b_e):
    BE = 4000
    grid = (E // BE,)
    return pl.pallas_call(
        _edge_mlp_body,
        grid=grid,
        in_specs=[
            pl.BlockSpec((BE, DE), lambda i: (i, 0)),
            pl.BlockSpec((DE, H), lambda i: (0, 0)),
            pl.BlockSpec((1, H), lambda i: (0, 0)),
        ],
        out_specs=pl.BlockSpec((BE, H), lambda i: (i, 0)),
        out_shape=jax.ShapeDtypeStruct((E_PAD, H), jnp.float32),
    )(edge_attr, w_e, b_e)


def _node_mlp_body(h_ref, part_ref, w1_ref, b1_ref, w2_ref, b2_ref,
                   lng_ref, lnb_ref, eps_ref, o_ref, *, residual):
    h = h_ref[...]
    agg = part_ref[0] + part_ref[1]
    z = (1.0 + eps_ref[0]) * h + agg
    z1 = jnp.dot(z, w1_ref[...], preferred_element_type=jnp.float32)
    z1 = jnp.maximum(z1 + b1_ref[...], 0.0)
    z2 = jnp.dot(z1, w2_ref[...], preferred_element_type=jnp.float32)
    z2 = z2 + b2_ref[...]
    mu = jnp.mean(z2, axis=-1, keepdims=True)
    var = jnp.mean((z2 - mu) ** 2, axis=-1, keepdims=True)
    zn = (z2 - mu) * lax.rsqrt(var + 1e-5) * lng_ref[...] + lnb_ref[...]
    zr = jnp.maximum(zn, 0.0)
    if residual:
        o_ref[...] = h + 0.3 * zr
    else:
        o_ref[...] = zr


def _node_mlp(h, part, w1, b1, w2, b2, lng, lnb, eps, residual):
    BN = 1000
    grid = (N // BN,)
    body = functools.partial(_node_mlp_body, residual=residual)
    return pl.pallas_call(
        body,
        grid=grid,
        in_specs=[
            pl.BlockSpec((BN, H), lambda i: (i, 0)),
            pl.BlockSpec((NC, BN, H), lambda i: (0, i, 0)),
            pl.BlockSpec((H, 2 * H), lambda i: (0, 0)),
            pl.BlockSpec((1, 2 * H), lambda i: (0, 0)),
            pl.BlockSpec((2 * H, H), lambda i: (0, 0)),
            pl.BlockSpec((1, H), lambda i: (0, 0)),
            pl.BlockSpec((1, H), lambda i: (0, 0)),
            pl.BlockSpec((1, H), lambda i: (0, 0)),
            pl.BlockSpec(memory_space=pltpu.SMEM),
        ],
        out_specs=pl.BlockSpec((BN, H), lambda i: (i, 0)),
        out_shape=jax.ShapeDtypeStruct((N, H), jnp.float32),
    )(h, part, w1, b1, w2, b2, lng, lnb, eps)


def kernel(x, edge_index, edge_attr,
           W_edge_0, b_edge_0, eps_0, W1_0, b1_0, bn1_g_0, bn1_b_0,
           W2_0, b2_0, bn_g_0, bn_b_0, ln_g_0, ln_b_0,
           W_edge_1, b_edge_1, eps_1, W1_1, b1_1, bn1_g_1, bn1_b_1,
           W2_1, b2_1, bn_g_1, bn_b_1, ln_g_1, ln_b_1,
           W_edge_2, b_edge_2, eps_2, W1_2, b1_2, bn1_g_2, bn1_b_2,
           W2_2, b2_2, bn_g_2, bn_b_2, ln_g_2, ln_b_2):
    bn_scale = 1.0 / jnp.sqrt(1.0 + 1e-5)
    pad_i = jnp.arange(E_PAD - E, dtype=jnp.int32)
    src_p = jnp.concatenate(
        [edge_index[0], pad_i % N]
    ).reshape(NC * NS, CHUNKS_PER_TILE, CHUNK)
    dst_p = jnp.concatenate(
        [edge_index[1], N + pad_i % (N_PAD - N)]
    ).reshape(NC * NS, CHUNKS_PER_TILE, CHUNK)
    sd_t = jnp.stack([src_p, dst_p], axis=2)
    zeros = jnp.zeros((N_PAD, H), jnp.float32)

    Ws, Es = [], []
    for (W_e, b_e, eps, W1, b1, g1, bb1, W2, b2, g2, bb2, lg, lb) in (
        (W_edge_0, b_edge_0, eps_0, W1_0, b1_0, bn1_g_0, bn1_b_0, W2_0, b2_0,
         bn_g_0, bn_b_0, ln_g_0, ln_b_0),
        (W_edge_1, b_edge_1, eps_1, W1_1, b1_1, bn1_g_1, bn1_b_1, W2_1, b2_1,
         bn_g_1, bn_b_1, ln_g_1, ln_b_1),
        (W_edge_2, b_edge_2, eps_2, W1_2, b1_2, bn1_g_2, bn1_b_2, W2_2, b2_2,
         bn_g_2, bn_b_2, ln_g_2, ln_b_2),
    ):
        s1 = bn_scale * g1
        w1f = W1 * s1[None, :]
        b1f = (b1 * s1 + bb1)[None, :]
        s2 = bn_scale * g2
        w2f = W2 * s2[None, :]
        b2f = (b2 * s2 + bb2)[None, :]
        Ws.append((eps.reshape(1), w1f, b1f, w2f, b2f,
                   lg[None, :], lb[None, :]))
        Es.append((W_e, b_e))

    h = x
    for i in range(3):
        eps, w1f, b1f, w2f, b2f, lg, lb = Ws[i]
        e = _edge_mlp(edge_attr, Es[i][0], Es[i][1][None, :])
        part = _sc_message_pass(h, e, sd_t, zeros)
        h = _node_mlp(h, part, w1f, b1f, w2f, b2f, lg, lb, eps,
                      residual=(i == 1))
    return h
